# baseline (speedup 1.0000x reference)
import functools

import jax
import jax.numpy as jnp
from jax import lax
from jax.experimental import pallas as pl
from jax.experimental.pallas import tpu as pltpu
from jax.experimental.pallas import tpu_sc as plsc

NC = 2
NS = 16
NW = NC * NS
L = 16
C = 80
W8 = 16


def _sc_mesh():
    return plsc.VectorSubcoreMesh(
        core_axis_name="c", subcore_axis_name="s", num_cores=NC,
        num_subcores=NS)


def _deg_body(nchunk, npt, row_hbm, ones_hbm, zero_hbm, out_hbm,
              i0, i1, i2, ones_v, acc, si0, si1, si2, ss0, ss1, ss2):
    idxc = (i0, i1, i2)
    sem_i = (si0, si1, si2)
    sem_s = (ss0, ss1, ss2)
    cid = lax.axis_index("c")
    sid = lax.axis_index("s")
    wid = sid * NC + cid
    pltpu.sync_copy(zero_hbm, acc.at[pl.ds(sid * npt, npt)])
    pltpu.sync_copy(ones_hbm, ones_v)
    plsc.subcore_barrier()

    def fetch_idx(j, s):
        base = (wid * nchunk + j) * C
        pltpu.async_copy(row_hbm.at[pl.ds(base, C)], idxc[s], sem_i[s])

    def wait_idx(j, s):
        base = (wid * nchunk + j) * C
        pltpu.make_async_copy(row_hbm.at[pl.ds(base, C)], idxc[s],
                              sem_i[s]).wait()

    def step(j, s, wait_prev, fetch_next):
        if wait_prev is True:
            pltpu.make_async_copy(ones_v, acc.at[idxc[(s + 1) % 3]],
                                  sem_s[(s + 1) % 3]).wait()
        elif wait_prev is not None and wait_prev is not False:
            pl.when(wait_prev)(
                lambda: pltpu.make_async_copy(
                    ones_v, acc.at[idxc[(s + 1) % 3]],
                    sem_s[(s + 1) % 3]).wait())
        if fetch_next:
            fetch_idx(j + 2, (s + 2) % 3)
        wait_idx(j, s)
        pltpu.async_copy(ones_v, acc.at[idxc[s]], sem_s[s], add=True)

    fetch_idx(0, 0)
    fetch_idx(1, 1)
    ntri = (nchunk - 2) // 3

    def tri(tt, carry):
        j0 = 3 * tt
        step(j0, 0, wait_prev=tt > 0, fetch_next=True)
        step(j0 + 1, 1, wait_prev=tt > 0, fetch_next=True)
        step(j0 + 2, 2, wait_prev=True, fetch_next=True)
        return carry

    lax.fori_loop(0, ntri, tri, 0)
    for j in range(3 * ntri, nchunk):
        step(j, j % 3, wait_prev=(j >= 2), fetch_next=(j + 2 < nchunk))
    pltpu.make_async_copy(ones_v, acc.at[idxc[(nchunk - 2) % 3]],
                          sem_s[(nchunk - 2) % 3]).wait()
    pltpu.make_async_copy(ones_v, acc.at[idxc[(nchunk - 1) % 3]],
                          sem_s[(nchunk - 1) % 3]).wait()
    plsc.subcore_barrier()
    pltpu.sync_copy(acc.at[pl.ds(sid * npt, npt)], out_hbm.at[cid, sid])


def _deg_kernel(n, rowf, nchunk):
    npt = n // NS
    ones = jnp.concatenate(
        [jnp.ones((C, 1), jnp.float32), jnp.zeros((C, 7), jnp.float32)], 1)
    zero = jnp.zeros((npt, 8), jnp.float32)
    body = functools.partial(_deg_body, nchunk, npt)
    fn = pl.kernel(
        body,
        out_type=jax.ShapeDtypeStruct((NC, NS, npt, 8), jnp.float32),
        mesh=_sc_mesh(),
        compiler_params=pltpu.CompilerParams(
            needs_layout_passes=False, use_tc_tiling_on_sc=False),
        scratch_types=(
            [pltpu.VMEM((C,), jnp.int32)] * 3
            + [pltpu.VMEM((C, 8), jnp.float32),
               pltpu.VMEM_SHARED((n, 8), jnp.float32)]
            + [pltpu.SemaphoreType.DMA] * 6
        ),
    )
    return fn(rowf, ones, zero)


def _layer_body(n, nchunk, npt, hs_hbm, row_hbm, col_hbm, a_hbm, b_hbm,
                zero_hbm, out_hbm, *scr):
    ridx = scr[0:3]
    cidx = scr[3:6]
    cs = scr[6:9]
    sa = scr[9:12]
    sb = scr[12:15]
    wch = scr[15]
    rows = scr[16:19]
    acc = scr[19]
    sem_i = scr[20:23]
    sem_g = scr[23:26]
    sem_s = scr[26:29]
    cid = lax.axis_index("c")
    sid = lax.axis_index("s")
    wid = sid * NC + cid
    pltpu.sync_copy(zero_hbm, acc.at[pl.ds(sid * npt, npt)])
    plsc.subcore_barrier()

    def fetch_idx(j, s):
        base = (wid * nchunk + j) * C
        pltpu.async_copy(row_hbm.at[pl.ds(base, C)], ridx[s], sem_i[s])
        pltpu.async_copy(col_hbm.at[pl.ds(base, C)], cidx[s], sem_i[s])

    def wait_idx(j, s):
        base = (wid * nchunk + j) * C
        pltpu.make_async_copy(row_hbm.at[pl.ds(base, C)], ridx[s],
                              sem_i[s]).wait()
        pltpu.make_async_copy(col_hbm.at[pl.ds(base, C)], cidx[s],
                              sem_i[s]).wait()

    def fire_gathers(s):
        pltpu.async_copy(hs_hbm.at[ridx[s]], rows[s], sem_g[s])
        pltpu.async_copy(a_hbm.at[ridx[s]], sa[s], sem_g[s])
        pltpu.async_copy(b_hbm.at[cidx[s]], sb[s], sem_g[s])

    def wait_gathers(s):
        pltpu.make_async_copy(hs_hbm.at[ridx[s]], rows[s],
                              sem_g[s]).wait()
        pltpu.make_async_copy(a_hbm.at[ridx[s]], sa[s], sem_g[s]).wait()
        pltpu.make_async_copy(b_hbm.at[cidx[s]], sb[s], sem_g[s]).wait()

    def wait_scatter(s):
        pltpu.make_async_copy(rows[s], acc.at[cs[s]], sem_s[s]).wait()

    def step(j, s, wait_prev, fetch_next, gather_next):
        if wait_prev is True:
            wait_scatter((s + 1) % 3)
        elif wait_prev is not None and wait_prev is not False:
            pl.when(wait_prev)(lambda: wait_scatter((s + 1) % 3))

        if fetch_next:
            fetch_idx(j + 2, (s + 2) % 3)
        if gather_next:
            wait_idx(j + 1, (s + 1) % 3)
            fire_gathers((s + 1) % 3)

        wait_gathers(s)
        for k in range(C // L):
            x = sa[s][pl.ds(k * L, L)] + sb[s][pl.ds(k * L, L)]
            t = jnp.exp(-2.0 * jnp.abs(x))
            wch[pl.ds(k * L, L)] = (
                jnp.sign(x) * (1.0 - t) / (1.0 + t))
            cs[s][pl.ds(k * L, L)] = cidx[s][pl.ds(k * L, L)]

        def scale(ei, carry2):
            w = plsc.load_gather(wch, [jnp.full((L,), ei, jnp.int32)])
            for v in range(8):
                rows[s][ei, pl.ds(v * L, L)] = (
                    rows[s][ei, pl.ds(v * L, L)] * w)
            return carry2

        lax.fori_loop(0, C, scale, 0, unroll=16)
        pltpu.async_copy(rows[s], acc.at[cs[s]], sem_s[s], add=True)

    fetch_idx(0, 0)
    fetch_idx(1, 1)
    wait_idx(0, 0)
    fire_gathers(0)

    ntri = (nchunk - 2) // 3

    def tri(tt, carry):
        j0 = 3 * tt
        step(j0, 0, wait_prev=tt > 0, fetch_next=True, gather_next=True)
        step(j0 + 1, 1, wait_prev=tt > 0, fetch_next=True,
             gather_next=True)
        step(j0 + 2, 2, wait_prev=True, fetch_next=True, gather_next=True)
        return carry

    lax.fori_loop(0, ntri, tri, 0)
    for j in range(3 * ntri, nchunk):
        step(j, j % 3, wait_prev=(j >= 2), fetch_next=(j + 2 < nchunk),
             gather_next=(j + 1 < nchunk))
    wait_scatter((nchunk - 2) % 3)
    wait_scatter((nchunk - 1) % 3)
    plsc.subcore_barrier()
    pltpu.sync_copy(acc.at[pl.ds(sid * npt, npt)], out_hbm.at[cid, sid])


def _layer_kernel(hs, rowf, colf, nchunk, a8, b8):
    n, d = hs.shape
    npt = n // NS
    zero = jnp.zeros((npt, d), jnp.float32)
    body = functools.partial(_layer_body, n, nchunk, npt)
    fn = pl.kernel(
        body,
        out_type=jax.ShapeDtypeStruct((NC, NS, npt, d), jnp.float32),
        mesh=_sc_mesh(),
        compiler_params=pltpu.CompilerParams(
            needs_layout_passes=False, use_tc_tiling_on_sc=False),
        scratch_types=(
            [pltpu.VMEM((C,), jnp.int32)] * 9
            + [pltpu.VMEM((C,), jnp.float32)] * 6
            + [pltpu.VMEM((C,), jnp.float32)]
            + [pltpu.VMEM((C, d), jnp.float32)] * 3
            + [pltpu.VMEM_SHARED((n, d), jnp.float32)]
            + [pltpu.SemaphoreType.DMA] * 9
        ),
    )
    return fn(hs, rowf, colf, a8, b8, zero)


def _tc_h1_body(h_ref, w1_ref, b1_ref, wab_ref, bg_ref,
                h1_ref, a_ref, b_ref):
    h1 = lax.dot_general(h_ref[...], w1_ref[...],
                         (((1,), (1,)), ((), ())),
                         preferred_element_type=jnp.float32)
    h1 = jnp.maximum(h1 + b1_ref[...][None, :], 0.0)
    h1_ref[...] = h1
    ab = lax.dot_general(h1, wab_ref[...], (((1,), (0,)), ((), ())),
                         preferred_element_type=jnp.float32)
    a_ref[...] = ab[:, 0:1] + bg_ref[0, 0]
    b_ref[...] = ab[:, 1:2]


def _tc_h1(h, w1, b1, wab, bg):
    n, d = h.shape
    return pl.pallas_call(
        _tc_h1_body,
        out_shape=[
            jax.ShapeDtypeStruct((n, d), jnp.float32),
            jax.ShapeDtypeStruct((n, 1), jnp.float32),
            jax.ShapeDtypeStruct((n, 1), jnp.float32),
        ],
    )(h, w1, b1, wab, bg)


def _tc_nd_body(h1_ref, degp_ref, hs_ref, nd_ref):
    deg = degp_ref[0, :, 0:1] + degp_ref[1, :, 0:1]
    nd = lax.rsqrt(jnp.maximum(deg, 1.0))
    nd_ref[...] = nd
    hs_ref[...] = nd * h1_ref[...]


def _tc_nd(h1, degp):
    n, d = h1.shape
    return pl.pallas_call(
        _tc_nd_body,
        out_shape=[
            jax.ShapeDtypeStruct((n, d), jnp.float32),
            jax.ShapeDtypeStruct((n, 1), jnp.float32),
        ],
    )(h1, degp)


def _tc_mid_body(h1_ref, part_ref, nd_ref, wab_ref, bg_ref,
                 hs_ref, a_ref, b_ref):
    nd = nd_ref[...]
    h2 = 0.3 * h1_ref[...] + nd * (part_ref[0] + part_ref[1])
    hs_ref[...] = nd * h2
    ab = lax.dot_general(h2, wab_ref[...], (((1,), (0,)), ((), ())),
                         preferred_element_type=jnp.float32)
    a_ref[...] = ab[:, 0:1] + bg_ref[0, 0]
    b_ref[...] = ab[:, 1:2]


def _tc_mid(h1, part, nd, wab, bg):
    n, d = h1.shape
    return pl.pallas_call(
        _tc_mid_body,
        out_shape=[
            jax.ShapeDtypeStruct((n, d), jnp.float32),
            jax.ShapeDtypeStruct((n, 1), jnp.float32),
            jax.ShapeDtypeStruct((n, 1), jnp.float32),
        ],
    )(h1, part, nd, wab, bg)


def _tc_fin_body(h1_ref, part_ref, nd_ref, out_ref):
    out_ref[...] = (0.3 * h1_ref[...]
                    + nd_ref[...] * (part_ref[0] + part_ref[1]))


def _tc_fin(h1, part, nd):
    n, d = h1.shape
    return pl.pallas_call(
        _tc_fin_body,
        out_shape=jax.ShapeDtypeStruct((n, d), jnp.float32),
    )(h1, part, nd)


def kernel(h, edge_index, W1, b1, Wg1, bg1, Wg2, bg2):
    n, d = h.shape
    e = edge_index.shape[1]
    epw = e // NW
    nchunk = epw // C
    rowf = edge_index[0]
    colf = edge_index[1]
    wab1 = jnp.stack([Wg1[0, :d], Wg1[0, d:]], axis=1)
    wab2 = jnp.stack([Wg2[0, :d], Wg2[0, d:]], axis=1)
    bg1m = bg1.reshape(1, 1)
    bg2m = bg2.reshape(1, 1)

    h1, a1, b1v = _tc_h1(h, W1, b1, wab1, bg1m)
    degp = _deg_kernel(n, rowf, nchunk).reshape(NC, n, 8)
    hs1, nd = _tc_nd(h1, degp)

    part1 = _layer_kernel(hs1, rowf, colf, nchunk, a1.reshape(n),
                          b1v.reshape(n)).reshape(NC, n, d)
    hs2, a2, b2v = _tc_mid(h1, part1, nd, wab2, bg2m)

    part2 = _layer_kernel(hs2, rowf, colf, nchunk, a2.reshape(n),
                          b2v.reshape(n)).reshape(NC, n, d)
    return _tc_fin(h1, part2, nd)


# speedup vs baseline: 1.0568x; 1.0568x over previous
import functools

import jax
import jax.numpy as jnp
from jax import lax
from jax.experimental import pallas as pl
from jax.experimental.pallas import tpu as pltpu
from jax.experimental.pallas import tpu_sc as plsc

NC = 2
NS = 16
NW = NC * NS
L = 16
C = 80
W8 = 16


def _sc_mesh():
    return plsc.VectorSubcoreMesh(
        core_axis_name="c", subcore_axis_name="s", num_cores=NC,
        num_subcores=NS)


def _deg_body(nchunk, npt, row_hbm, ones_hbm, zero_hbm, out_hbm,
              i0, i1, i2, ones_v, acc, si0, si1, si2, ss0, ss1, ss2):
    idxc = (i0, i1, i2)
    sem_i = (si0, si1, si2)
    sem_s = (ss0, ss1, ss2)
    cid = lax.axis_index("c")
    sid = lax.axis_index("s")
    wid = sid * NC + cid
    pltpu.sync_copy(zero_hbm, acc.at[pl.ds(sid * npt, npt)])
    pltpu.sync_copy(ones_hbm, ones_v)
    plsc.subcore_barrier()

    def fetch_idx(j, s):
        base = (wid * nchunk + j) * C
        pltpu.async_copy(row_hbm.at[pl.ds(base, C)], idxc[s], sem_i[s])

    def wait_idx(j, s):
        base = (wid * nchunk + j) * C
        pltpu.make_async_copy(row_hbm.at[pl.ds(base, C)], idxc[s],
                              sem_i[s]).wait()

    def step(j, s, wait_prev, fetch_next):
        if wait_prev is True:
            pltpu.make_async_copy(ones_v, acc.at[idxc[(s + 1) % 3]],
                                  sem_s[(s + 1) % 3]).wait()
        elif wait_prev is not None and wait_prev is not False:
            pl.when(wait_prev)(
                lambda: pltpu.make_async_copy(
                    ones_v, acc.at[idxc[(s + 1) % 3]],
                    sem_s[(s + 1) % 3]).wait())
        if fetch_next:
            fetch_idx(j + 2, (s + 2) % 3)
        wait_idx(j, s)
        pltpu.async_copy(ones_v, acc.at[idxc[s]], sem_s[s], add=True)

    fetch_idx(0, 0)
    fetch_idx(1, 1)
    ntri = (nchunk - 2) // 3

    def tri(tt, carry):
        j0 = 3 * tt
        step(j0, 0, wait_prev=tt > 0, fetch_next=True)
        step(j0 + 1, 1, wait_prev=tt > 0, fetch_next=True)
        step(j0 + 2, 2, wait_prev=True, fetch_next=True)
        return carry

    lax.fori_loop(0, ntri, tri, 0)
    for j in range(3 * ntri, nchunk):
        step(j, j % 3, wait_prev=(j >= 2), fetch_next=(j + 2 < nchunk))
    pltpu.make_async_copy(ones_v, acc.at[idxc[(nchunk - 2) % 3]],
                          sem_s[(nchunk - 2) % 3]).wait()
    pltpu.make_async_copy(ones_v, acc.at[idxc[(nchunk - 1) % 3]],
                          sem_s[(nchunk - 1) % 3]).wait()
    plsc.subcore_barrier()
    pltpu.sync_copy(acc.at[pl.ds(sid * npt, npt)], out_hbm.at[cid, sid])


def _deg_kernel(n, rowf, nchunk):
    npt = n // NS
    ones = jnp.concatenate(
        [jnp.ones((C, 1), jnp.float32), jnp.zeros((C, 7), jnp.float32)], 1)
    zero = jnp.zeros((npt, 8), jnp.float32)
    body = functools.partial(_deg_body, nchunk, npt)
    fn = pl.kernel(
        body,
        out_type=jax.ShapeDtypeStruct((NC, NS, npt, 8), jnp.float32),
        mesh=_sc_mesh(),
        compiler_params=pltpu.CompilerParams(
            needs_layout_passes=False, use_tc_tiling_on_sc=False),
        scratch_types=(
            [pltpu.VMEM((C,), jnp.int32)] * 3
            + [pltpu.VMEM((C, 8), jnp.float32),
               pltpu.VMEM_SHARED((n, 8), jnp.float32)]
            + [pltpu.SemaphoreType.DMA] * 6
        ),
    )
    return fn(rowf, ones, zero)


def _layer_body(n, nchunk, npt, hs_hbm, row_hbm, col_hbm, a_hbm, b_hbm,
                zero_hbm, out_hbm, *scr):
    ridx = scr[0:3]
    cidx = scr[3:6]
    cs = scr[6:9]
    sa = scr[9:12]
    sb = scr[12:15]
    wch = scr[15]
    rows = scr[16:19]
    acc = scr[19]
    sem_i = scr[20:23]
    sem_g = scr[23:26]
    sem_h = scr[26:29]
    sem_s = scr[29:32]
    cid = lax.axis_index("c")
    sid = lax.axis_index("s")
    wid = sid * NC + cid
    pltpu.sync_copy(zero_hbm, acc.at[pl.ds(sid * npt, npt)])
    plsc.subcore_barrier()

    def fetch_idx(j, s):
        base = (wid * nchunk + j) * C
        pltpu.async_copy(row_hbm.at[pl.ds(base, C)], ridx[s], sem_i[s])
        pltpu.async_copy(col_hbm.at[pl.ds(base, C)], cidx[s], sem_i[s])

    def wait_idx(j, s):
        base = (wid * nchunk + j) * C
        pltpu.make_async_copy(row_hbm.at[pl.ds(base, C)], ridx[s],
                              sem_i[s]).wait()
        pltpu.make_async_copy(col_hbm.at[pl.ds(base, C)], cidx[s],
                              sem_i[s]).wait()

    def fire_gathers(s):
        pltpu.async_copy(hs_hbm.at[ridx[s]], rows[s], sem_g[s])
        pltpu.async_copy(a_hbm.at[ridx[s]], sa[s], sem_h[s])
        pltpu.async_copy(b_hbm.at[cidx[s]], sb[s], sem_h[s])

    def wait_scalars(s):
        pltpu.make_async_copy(a_hbm.at[ridx[s]], sa[s], sem_h[s]).wait()
        pltpu.make_async_copy(b_hbm.at[cidx[s]], sb[s], sem_h[s]).wait()

    def wait_rows(s):
        pltpu.make_async_copy(hs_hbm.at[ridx[s]], rows[s],
                              sem_g[s]).wait()

    def wait_scatter(s):
        pltpu.make_async_copy(rows[s], acc.at[cs[s]], sem_s[s]).wait()

    def step(j, s, wait_prev, fetch_next, gather_next):
        if wait_prev is True:
            wait_scatter((s + 1) % 3)
        elif wait_prev is not None and wait_prev is not False:
            pl.when(wait_prev)(lambda: wait_scatter((s + 1) % 3))

        if fetch_next:
            fetch_idx(j + 2, (s + 2) % 3)
        if gather_next:
            wait_idx(j + 1, (s + 1) % 3)
            fire_gathers((s + 1) % 3)

        wait_scalars(s)
        for k in range(C // L):
            x = sa[s][pl.ds(k * L, L)] + sb[s][pl.ds(k * L, L)]
            t = jnp.exp(-2.0 * jnp.abs(x))
            wch[pl.ds(k * L, L)] = (
                jnp.sign(x) * (1.0 - t) / (1.0 + t))
            cs[s][pl.ds(k * L, L)] = cidx[s][pl.ds(k * L, L)]

        wait_rows(s)

        def scale(ei, carry2):
            w = plsc.load_gather(wch, [jnp.full((L,), ei, jnp.int32)])
            for v in range(8):
                rows[s][ei, pl.ds(v * L, L)] = (
                    rows[s][ei, pl.ds(v * L, L)] * w)
            return carry2

        lax.fori_loop(0, C, scale, 0, unroll=8)
        pltpu.async_copy(rows[s], acc.at[cs[s]], sem_s[s], add=True)

    fetch_idx(0, 0)
    fetch_idx(1, 1)
    wait_idx(0, 0)
    fire_gathers(0)

    ntri = (nchunk - 2) // 3

    def tri(tt, carry):
        j0 = 3 * tt
        step(j0, 0, wait_prev=tt > 0, fetch_next=True, gather_next=True)
        step(j0 + 1, 1, wait_prev=tt > 0, fetch_next=True,
             gather_next=True)
        step(j0 + 2, 2, wait_prev=True, fetch_next=True, gather_next=True)
        return carry

    lax.fori_loop(0, ntri, tri, 0)
    for j in range(3 * ntri, nchunk):
        step(j, j % 3, wait_prev=(j >= 2), fetch_next=(j + 2 < nchunk),
             gather_next=(j + 1 < nchunk))
    wait_scatter((nchunk - 2) % 3)
    wait_scatter((nchunk - 1) % 3)
    plsc.subcore_barrier()
    pltpu.sync_copy(acc.at[pl.ds(sid * npt, npt)], out_hbm.at[cid, sid])


def _layer_kernel(hs, rowf, colf, nchunk, a8, b8):
    n, d = hs.shape
    npt = n // NS
    zero = jnp.zeros((npt, d), jnp.float32)
    body = functools.partial(_layer_body, n, nchunk, npt)
    fn = pl.kernel(
        body,
        out_type=jax.ShapeDtypeStruct((NC, NS, npt, d), jnp.float32),
        mesh=_sc_mesh(),
        compiler_params=pltpu.CompilerParams(
            needs_layout_passes=False, use_tc_tiling_on_sc=False),
        scratch_types=(
            [pltpu.VMEM((C,), jnp.int32)] * 9
            + [pltpu.VMEM((C,), jnp.float32)] * 6
            + [pltpu.VMEM((C,), jnp.float32)]
            + [pltpu.VMEM((C, d), jnp.float32)] * 3
            + [pltpu.VMEM_SHARED((n, d), jnp.float32)]
            + [pltpu.SemaphoreType.DMA] * 12
        ),
    )
    return fn(hs, rowf, colf, a8, b8, zero)


def _tc_h1_body(h_ref, w1_ref, b1_ref, wab_ref, bg_ref,
                h1_ref, a_ref, b_ref):
    h1 = lax.dot_general(h_ref[...], w1_ref[...],
                         (((1,), (1,)), ((), ())),
                         preferred_element_type=jnp.float32)
    h1 = jnp.maximum(h1 + b1_ref[...][None, :], 0.0)
    h1_ref[...] = h1
    ab = lax.dot_general(h1, wab_ref[...], (((1,), (0,)), ((), ())),
                         preferred_element_type=jnp.float32)
    a_ref[...] = ab[:, 0:1] + bg_ref[0, 0]
    b_ref[...] = ab[:, 1:2]


def _tc_h1(h, w1, b1, wab, bg):
    n, d = h.shape
    return pl.pallas_call(
        _tc_h1_body,
        out_shape=[
            jax.ShapeDtypeStruct((n, d), jnp.float32),
            jax.ShapeDtypeStruct((n, 1), jnp.float32),
            jax.ShapeDtypeStruct((n, 1), jnp.float32),
        ],
    )(h, w1, b1, wab, bg)


def _tc_nd_body(h1_ref, degp_ref, hs_ref, nd_ref):
    deg = degp_ref[0, :, 0:1] + degp_ref[1, :, 0:1]
    nd = lax.rsqrt(jnp.maximum(deg, 1.0))
    nd_ref[...] = nd
    hs_ref[...] = nd * h1_ref[...]


def _tc_nd(h1, degp):
    n, d = h1.shape
    return pl.pallas_call(
        _tc_nd_body,
        out_shape=[
            jax.ShapeDtypeStruct((n, d), jnp.float32),
            jax.ShapeDtypeStruct((n, 1), jnp.float32),
        ],
    )(h1, degp)


def _tc_mid_body(h1_ref, part_ref, nd_ref, wab_ref, bg_ref,
                 hs_ref, a_ref, b_ref):
    nd = nd_ref[...]
    h2 = 0.3 * h1_ref[...] + nd * (part_ref[0] + part_ref[1])
    hs_ref[...] = nd * h2
    ab = lax.dot_general(h2, wab_ref[...], (((1,), (0,)), ((), ())),
                         preferred_element_type=jnp.float32)
    a_ref[...] = ab[:, 0:1] + bg_ref[0, 0]
    b_ref[...] = ab[:, 1:2]


def _tc_mid(h1, part, nd, wab, bg):
    n, d = h1.shape
    return pl.pallas_call(
        _tc_mid_body,
        out_shape=[
            jax.ShapeDtypeStruct((n, d), jnp.float32),
            jax.ShapeDtypeStruct((n, 1), jnp.float32),
            jax.ShapeDtypeStruct((n, 1), jnp.float32),
        ],
    )(h1, part, nd, wab, bg)


def _tc_fin_body(h1_ref, part_ref, nd_ref, out_ref):
    out_ref[...] = (0.3 * h1_ref[...]
                    + nd_ref[...] * (part_ref[0] + part_ref[1]))


def _tc_fin(h1, part, nd):
    n, d = h1.shape
    return pl.pallas_call(
        _tc_fin_body,
        out_shape=jax.ShapeDtypeStruct((n, d), jnp.float32),
    )(h1, part, nd)


def kernel(h, edge_index, W1, b1, Wg1, bg1, Wg2, bg2):
    n, d = h.shape
    e = edge_index.shape[1]
    epw = e // NW
    nchunk = epw // C
    rowf = edge_index[0]
    colf = edge_index[1]
    wab1 = jnp.stack([Wg1[0, :d], Wg1[0, d:]], axis=1)
    wab2 = jnp.stack([Wg2[0, :d], Wg2[0, d:]], axis=1)
    bg1m = bg1.reshape(1, 1)
    bg2m = bg2.reshape(1, 1)

    h1, a1, b1v = _tc_h1(h, W1, b1, wab1, bg1m)
    degp = _deg_kernel(n, rowf, nchunk).reshape(NC, n, 8)
    hs1, nd = _tc_nd(h1, degp)

    part1 = _layer_kernel(hs1, rowf, colf, nchunk, a1.reshape(n),
                          b1v.reshape(n)).reshape(NC, n, d)
    hs2, a2, b2v = _tc_mid(h1, part1, nd, wab2, bg2m)

    part2 = _layer_kernel(hs2, rowf, colf, nchunk, a2.reshape(n),
                          b2v.reshape(n)).reshape(NC, n, d)
    return _tc_fin(h1, part2, nd)
